# baseline (speedup 1.0000x reference)
import jax
import jax.numpy as jnp
from jax import lax
from jax.experimental import pallas as pl
from jax.experimental.pallas import tpu as pltpu

D = 128
TILE = 9984

_OFF_A = 0
_OFF_T = 504
_OFF_S = 520
_OFF_TF = 544
_ROWS = 552

_DN_B1 = (((0,), (1,)), ((), ()))
_DN_STD = (((1,), (0,)), ((), ()))
_DN_RT = (((1,), (1,)), ((), ()))
_DN_COL = (((0,), (0,)), ((), ()))


def _prep_body(w2_ref, b1r_ref, b2r_ref,
               wat_ref, bar_ref, wtt_ref, btr_ref,
               wst_ref, bsr_ref, wtft_ref, btfr_ref,
               wall_ref, ball_ref, b1c_ref):
    w2 = w2_ref[...]
    b2r = b2r_ref[...]
    wall_ref[...] = jnp.zeros((_ROWS, D), jnp.float32)
    ball_ref[...] = jnp.zeros((_ROWS, 1), jnp.float32)

    def fold(wct_ref, bcr_ref, off, rows):
        wct = wct_ref[...]
        wall_ref[pl.ds(off, rows), :] = lax.dot_general(
            wct, w2, _DN_RT, preferred_element_type=jnp.float32)
        brow = lax.dot_general(b2r, wct, _DN_RT,
                               preferred_element_type=jnp.float32) + bcr_ref[...]
        ball_ref[pl.ds(off, rows), :] = brow.T

    fold(wat_ref, bar_ref, _OFF_A, 500)
    fold(wtt_ref, btr_ref, _OFF_T, 10)
    fold(wst_ref, bsr_ref, _OFF_S, 20)
    fold(wtft_ref, btfr_ref, _OFF_TF, 2)
    b1c_ref[...] = b1r_ref[...].T


def _body(x_ref, w1_ref, b1c_ref, wall_ref, ball_ref,
          ot_ref, os_ref, otf_ref, oa_ref):
    hT = jnp.maximum(
        lax.dot_general(w1_ref[...].astype(jnp.bfloat16),
                        x_ref[...].astype(jnp.bfloat16), _DN_B1,
                        preferred_element_type=jnp.float32) + b1c_ref[...],
        0.0)
    res = lax.dot_general(wall_ref[...].astype(jnp.bfloat16),
                          hT.astype(jnp.bfloat16), _DN_STD,
                          preferred_element_type=jnp.float32) + ball_ref[...]
    oa_ref[...] = res[_OFF_A:_OFF_A + 500, :]
    ot_ref[...] = res[_OFF_T:_OFF_T + 10, :]
    os_ref[...] = res[_OFF_S:_OFF_S + 20, :]
    otf_ref[...] = res[_OFF_TF:_OFF_TF + 2, :]


@jax.jit
def kernel(x, W1, b1, W2, b2, Wt, bt, Ws, bs, Wtf, btf, Wa, ba):
    n = x.shape[0]

    def full(a):
        return pl.BlockSpec(a.shape, lambda *_: (0,) * a.ndim)

    b1r = b1.reshape(1, -1)
    b2r = b2.reshape(1, -1)
    prep_in = (W2, b1r, b2r,
               Wa.T, ba.reshape(1, -1), Wt.T, bt.reshape(1, -1),
               Ws.T, bs.reshape(1, -1), Wtf.T, btf.reshape(1, -1))
    Wall, ball, b1c = pl.pallas_call(
        _prep_body,
        grid=(1,),
        in_specs=[full(a) for a in prep_in],
        out_specs=(
            pl.BlockSpec((_ROWS, D), lambda i: (0, 0)),
            pl.BlockSpec((_ROWS, 1), lambda i: (0, 0)),
            pl.BlockSpec((D, 1), lambda i: (0, 0)),
        ),
        out_shape=(
            jax.ShapeDtypeStruct((_ROWS, D), jnp.float32),
            jax.ShapeDtypeStruct((_ROWS, 1), jnp.float32),
            jax.ShapeDtypeStruct((D, 1), jnp.float32),
        ),
    )(*prep_in)

    col = lambda c: pl.BlockSpec((c, TILE), lambda i: (0, i))

    otT, osT, otfT, oaT = pl.pallas_call(
        _body,
        grid=(pl.cdiv(n, TILE),),
        in_specs=[
            pl.BlockSpec((TILE, D), lambda i: (i, 0)),
            full(W1), full(b1c), full(Wall), full(ball),
        ],
        out_specs=(
            col(Wt.shape[1]), col(Ws.shape[1]),
            col(Wtf.shape[1]), col(Wa.shape[1]),
        ),
        out_shape=(
            jax.ShapeDtypeStruct((Wt.shape[1], n), jnp.float32),
            jax.ShapeDtypeStruct((Ws.shape[1], n), jnp.float32),
            jax.ShapeDtypeStruct((Wtf.shape[1], n), jnp.float32),
            jax.ShapeDtypeStruct((Wa.shape[1], n), jnp.float32),
        ),
        compiler_params=pltpu.CompilerParams(
            dimension_semantics=("parallel",),
        ),
    )(x, W1, b1c, Wall, ball)

    return (otT.T, osT.T, otfT.T, oaT.T)


# speedup vs baseline: 1.0221x; 1.0221x over previous
import jax
import jax.numpy as jnp
from jax import lax
from jax.experimental import pallas as pl
from jax.experimental.pallas import tpu as pltpu

D = 128
TILE = 9984

_OFF_A = 0
_OFF_T = 504
_OFF_S = 520
_OFF_TF = 544
_ROWS = 552

_DN_B1 = (((0,), (1,)), ((), ()))
_DN_STD = (((1,), (0,)), ((), ()))
_DN_RT = (((1,), (1,)), ((), ()))

_HEADS = ((_OFF_A, 500), (_OFF_T, 10), (_OFF_S, 20), (_OFF_TF, 2))


def _body(x_ref, w1_ref, b1r_ref, w2_ref, b2r_ref,
          wat_ref, bar_ref, wtt_ref, btr_ref,
          wst_ref, bsr_ref, wtft_ref, btfr_ref,
          ot_ref, os_ref, otf_ref, oa_ref,
          wall_ref, ball_ref, b1c_ref):

    @pl.when(pl.program_id(0) == 0)
    def _fold_weights():
        w2 = w2_ref[...]
        b2r = b2r_ref[...]
        wall_ref[...] = jnp.zeros((_ROWS, D), jnp.float32)
        ball_ref[...] = jnp.zeros((_ROWS, 1), jnp.float32)

        def fold(wct_ref, bcr_ref, off, rows):
            wct = wct_ref[...]
            wall_ref[pl.ds(off, rows), :] = lax.dot_general(
                wct, w2, _DN_RT, preferred_element_type=jnp.float32)
            brow = lax.dot_general(b2r, wct, _DN_RT,
                                   preferred_element_type=jnp.float32) + bcr_ref[...]
            ball_ref[pl.ds(off, rows), :] = brow.T

        fold(wat_ref, bar_ref, _OFF_A, 500)
        fold(wtt_ref, btr_ref, _OFF_T, 10)
        fold(wst_ref, bsr_ref, _OFF_S, 20)
        fold(wtft_ref, btfr_ref, _OFF_TF, 2)
        b1c_ref[...] = b1r_ref[...].T

    hT = jnp.maximum(
        lax.dot_general(w1_ref[...], x_ref[...], _DN_B1,
                        preferred_element_type=jnp.float32) + b1c_ref[...],
        0.0)
    res = lax.dot_general(wall_ref[...], hT, _DN_STD,
                          preferred_element_type=jnp.float32) + ball_ref[...]
    oa_ref[...] = res[_OFF_A:_OFF_A + 500, :]
    ot_ref[...] = res[_OFF_T:_OFF_T + 10, :]
    os_ref[...] = res[_OFF_S:_OFF_S + 20, :]
    otf_ref[...] = res[_OFF_TF:_OFF_TF + 2, :]


@jax.jit
def kernel(x, W1, b1, W2, b2, Wt, bt, Ws, bs, Wtf, btf, Wa, ba):
    n = x.shape[0]

    def full(a):
        return pl.BlockSpec(a.shape, lambda *_: (0,) * a.ndim)

    weights = (W1, b1.reshape(1, -1), W2, b2.reshape(1, -1),
               Wa.T, ba.reshape(1, -1), Wt.T, bt.reshape(1, -1),
               Ws.T, bs.reshape(1, -1), Wtf.T, btf.reshape(1, -1))

    col = lambda c: pl.BlockSpec((c, TILE), lambda i: (0, i))

    otT, osT, otfT, oaT = pl.pallas_call(
        _body,
        grid=(pl.cdiv(n, TILE),),
        in_specs=[pl.BlockSpec((TILE, D), lambda i: (i, 0))]
        + [full(a) for a in weights],
        out_specs=(
            col(Wt.shape[1]), col(Ws.shape[1]),
            col(Wtf.shape[1]), col(Wa.shape[1]),
        ),
        out_shape=(
            jax.ShapeDtypeStruct((Wt.shape[1], n), jnp.float32),
            jax.ShapeDtypeStruct((Ws.shape[1], n), jnp.float32),
            jax.ShapeDtypeStruct((Wtf.shape[1], n), jnp.float32),
            jax.ShapeDtypeStruct((Wa.shape[1], n), jnp.float32),
        ),
        scratch_shapes=[
            pltpu.VMEM((_ROWS, D), jnp.float32),
            pltpu.VMEM((_ROWS, 1), jnp.float32),
            pltpu.VMEM((D, 1), jnp.float32),
        ],
        compiler_params=pltpu.CompilerParams(
            dimension_semantics=("arbitrary",),
        ),
    )(x, *weights)

    return (otT.T, osT.T, otfT.T, oaT.T)
